# Initial kernel scaffold; baseline (speedup 1.0000x reference)
import jax
import jax.numpy as jnp
from jax.experimental import pallas as pl


def kernel(x_num, x_wide, x_cat, tables, W1, b1, g1, beta1, W2, b2, g2, beta2, W3, b3, Ww, bw):
    raise NotImplementedError("write your pallas kernel here")


# speedup vs baseline: 6.9487x; 6.9487x over previous
import functools

import jax
import jax.numpy as jnp
from jax import lax
from jax.experimental import pallas as pl
from jax.experimental.pallas import tpu as pltpu
from jax.experimental.pallas import tpu_sc as plsc

B = 16384
F = 26
V = 100000
D = 16
NUM = 13
WIDE = 1000
H1 = 128
H2 = 64
EPS = 1e-5

NC, NS = 2, 16
NW = NC * NS
RPW = (B * F) // NW
NCH = 8
CH = RPW // NCH

BB = 2048
NB = B // BB

_mesh = plsc.VectorSubcoreMesh(core_axis_name="c", subcore_axis_name="s")


@functools.partial(
    pl.kernel,
    mesh=_mesh,
    compiler_params=pltpu.CompilerParams(use_tc_tiling_on_sc=False),
    out_type=jax.ShapeDtypeStruct((B * F, D), jnp.float32),
    scratch_types=[
        pltpu.VMEM((RPW,), jnp.int32),
        pltpu.VMEM((CH, D), jnp.float32),
        pltpu.SemaphoreType.DMA,
    ],
)
def _sc_gather(tab_hbm, idx_hbm, out_hbm, idx_v, rows_v, sem):
    wid = lax.axis_index("s") * NC + lax.axis_index("c")
    base = wid * RPW
    pltpu.sync_copy(idx_hbm.at[pl.ds(base, RPW)], idx_v)
    for j in range(NCH):
        pltpu.async_copy(
            tab_hbm.at[idx_v.at[pl.ds(j * CH, CH)]], rows_v, sem
        ).wait()
        pltpu.sync_copy(rows_v, out_hbm.at[pl.ds(base + j * CH, CH)])


_P = lax.Precision.HIGHEST


def _k1(xe, xn, xw, w1e, w1n, b1, ww, h1o, s1o, ss1o, wlo):
    h = (
        jnp.dot(xe[...], w1e[...], preferred_element_type=jnp.float32, precision=_P)
        + jnp.dot(xn[...], w1n[...], preferred_element_type=jnp.float32, precision=_P)
        + b1[...]
    )
    h1o[...] = h
    s1o[...] = jnp.sum(h, axis=0, keepdims=True)[None]
    ss1o[...] = jnp.sum(h * h, axis=0, keepdims=True)[None]
    wlo[...] = jnp.dot(xw[...], ww[...], preferred_element_type=jnp.float32, precision=_P)


def _k2(h1, s1, ss1, g1, beta1, w2, b2, h2o, s2o, ss2o):
    inv_b = jnp.float32(1.0 / B)
    mean = jnp.sum(s1[...], axis=0) * inv_b
    m2 = jnp.sum(ss1[...], axis=0) * inv_b
    var = m2 - mean * mean
    rstd = lax.rsqrt(var + EPS)
    hn = jnp.maximum((h1[...] - mean) * (rstd * g1[...]) + beta1[...], 0.0)
    h2 = jnp.dot(hn, w2[...], preferred_element_type=jnp.float32, precision=_P) + b2[...]
    h2o[...] = h2
    s2o[...] = jnp.sum(h2, axis=0, keepdims=True)[None]
    ss2o[...] = jnp.sum(h2 * h2, axis=0, keepdims=True)[None]


def _k3(h2, s2, ss2, g2, beta2, w3, b3, bw, wl, out):
    inv_b = jnp.float32(1.0 / B)
    mean = jnp.sum(s2[...], axis=0) * inv_b
    m2 = jnp.sum(ss2[...], axis=0) * inv_b
    var = m2 - mean * mean
    rstd = lax.rsqrt(var + EPS)
    hn = jnp.maximum((h2[...] - mean) * (rstd * g2[...]) + beta2[...], 0.0)
    out[...] = (
        jnp.dot(hn, w3[...], preferred_element_type=jnp.float32, precision=_P)
        + b3[...]
        + bw[...]
        + wl[...]
    )


def kernel(x_num, x_wide, x_cat, tables, W1, b1, g1, beta1, W2, b2, g2, beta2, W3, b3, Ww, bw):
    tab_flat = tables.reshape(F * V, D)
    flat_idx = (x_cat + (jnp.arange(F, dtype=jnp.int32) * V)[None, :]).reshape(B * F)
    x_emb = _sc_gather(tab_flat, flat_idx).reshape(B, F * D)

    w1e = W1[:, : F * D].T
    w1n = W1[:, F * D :].T

    h1, s1, ss1, wl = pl.pallas_call(
        _k1,
        grid=(NB,),
        in_specs=[
            pl.BlockSpec((BB, F * D), lambda i: (i, 0)),
            pl.BlockSpec((BB, NUM), lambda i: (i, 0)),
            pl.BlockSpec((BB, WIDE), lambda i: (i, 0)),
            pl.BlockSpec((F * D, H1), lambda i: (0, 0)),
            pl.BlockSpec((NUM, H1), lambda i: (0, 0)),
            pl.BlockSpec((1, H1), lambda i: (0, 0)),
            pl.BlockSpec((WIDE, 1), lambda i: (0, 0)),
        ],
        out_specs=[
            pl.BlockSpec((BB, H1), lambda i: (i, 0)),
            pl.BlockSpec((1, 1, H1), lambda i: (i, 0, 0)),
            pl.BlockSpec((1, 1, H1), lambda i: (i, 0, 0)),
            pl.BlockSpec((BB, 1), lambda i: (i, 0)),
        ],
        out_shape=[
            jax.ShapeDtypeStruct((B, H1), jnp.float32),
            jax.ShapeDtypeStruct((NB, 1, H1), jnp.float32),
            jax.ShapeDtypeStruct((NB, 1, H1), jnp.float32),
            jax.ShapeDtypeStruct((B, 1), jnp.float32),
        ],
    )(x_emb, x_num, x_wide, w1e, w1n, b1.reshape(1, H1), Ww.T)

    h2, s2, ss2 = pl.pallas_call(
        _k2,
        grid=(NB,),
        in_specs=[
            pl.BlockSpec((BB, H1), lambda i: (i, 0)),
            pl.BlockSpec((NB, 1, H1), lambda i: (0, 0, 0)),
            pl.BlockSpec((NB, 1, H1), lambda i: (0, 0, 0)),
            pl.BlockSpec((1, H1), lambda i: (0, 0)),
            pl.BlockSpec((1, H1), lambda i: (0, 0)),
            pl.BlockSpec((H1, H2), lambda i: (0, 0)),
            pl.BlockSpec((1, H2), lambda i: (0, 0)),
        ],
        out_specs=[
            pl.BlockSpec((BB, H2), lambda i: (i, 0)),
            pl.BlockSpec((1, 1, H2), lambda i: (i, 0, 0)),
            pl.BlockSpec((1, 1, H2), lambda i: (i, 0, 0)),
        ],
        out_shape=[
            jax.ShapeDtypeStruct((B, H2), jnp.float32),
            jax.ShapeDtypeStruct((NB, 1, H2), jnp.float32),
            jax.ShapeDtypeStruct((NB, 1, H2), jnp.float32),
        ],
    )(h1, s1, ss1, g1.reshape(1, H1), beta1.reshape(1, H1), W2.T, b2.reshape(1, H2))

    out = pl.pallas_call(
        _k3,
        grid=(NB,),
        in_specs=[
            pl.BlockSpec((BB, H2), lambda i: (i, 0)),
            pl.BlockSpec((NB, 1, H2), lambda i: (0, 0, 0)),
            pl.BlockSpec((NB, 1, H2), lambda i: (0, 0, 0)),
            pl.BlockSpec((1, H2), lambda i: (0, 0)),
            pl.BlockSpec((1, H2), lambda i: (0, 0)),
            pl.BlockSpec((H2, 1), lambda i: (0, 0)),
            pl.BlockSpec((1, 1), lambda i: (0, 0)),
            pl.BlockSpec((1, 1), lambda i: (0, 0)),
            pl.BlockSpec((BB, 1), lambda i: (i, 0)),
        ],
        out_specs=pl.BlockSpec((BB, 1), lambda i: (i, 0)),
        out_shape=jax.ShapeDtypeStruct((B, 1), jnp.float32),
    )(h2, s2, ss2, g2.reshape(1, H2), beta2.reshape(1, H2), W3.T,
      b3.reshape(1, 1), bw.reshape(1, 1), wl)

    return out[:, 0]


# speedup vs baseline: 18.3834x; 2.6456x over previous
import functools

import jax
import jax.numpy as jnp
from jax import lax
from jax.experimental import pallas as pl
from jax.experimental.pallas import tpu as pltpu
from jax.experimental.pallas import tpu_sc as plsc

B = 16384
F = 26
V = 100000
D = 16
NUM = 13
WIDE = 1000
H1 = 128
H2 = 64
EPS = 1e-5

NC, NS = 2, 16
NW = NC * NS
NPL = (F * D) // NW
HALF = B // 2

BB = 2048
NB = B // BB

_mesh = plsc.VectorSubcoreMesh(core_axis_name="c", subcore_axis_name="s")


@functools.partial(
    pl.kernel,
    mesh=_mesh,
    compiler_params=pltpu.CompilerParams(
        use_tc_tiling_on_sc=False, needs_layout_passes=False
    ),
    out_type=jax.ShapeDtypeStruct((F * D, B), jnp.float32),
    scratch_types=[
        pltpu.VMEM((V,), jnp.float32),
        pltpu.VMEM((HALF,), jnp.int32),
        pltpu.VMEM((HALF,), jnp.float32),
        pltpu.SemaphoreType.DMA,
    ],
)
def _sc_plane_gather(tab_hbm, xcat_hbm, out_hbm, plane_v, idx_v, out_v, sem):
    wid = lax.axis_index("s") * NC + lax.axis_index("c")
    p0 = wid * NPL

    def per_plane(j, _):
        p = p0 + j
        f = p // D
        pltpu.async_copy(tab_hbm.at[p], plane_v, sem).wait()
        for h in range(2):
            pltpu.sync_copy(xcat_hbm.at[f, pl.ds(h * HALF, HALF)], idx_v)

            def per_row(r, _):
                for c in range(8):
                    k = r * 128 + c * 16
                    vals = plsc.load_gather(plane_v, [idx_v[pl.ds(k, 16)]])
                    out_v[pl.ds(k, 16)] = vals
                return 0

            lax.fori_loop(0, HALF // 128, per_row, 0)
            pltpu.sync_copy(out_v, out_hbm.at[p, pl.ds(h * HALF, HALF)])
        return 0

    lax.fori_loop(0, NPL, per_plane, 0)


def _k1(xe3, xnt, xwt, w1eT, w1nT, b1c, ww, h1o, s1o, ss1o, wlo):
    xe = xe3[...]
    h = (
        lax.dot_general(w1eT[...], xe, (((0,), (0,)), ((), ())),
                        preferred_element_type=jnp.float32)
        + lax.dot_general(w1nT[...], xnt[...], (((0,), (0,)), ((), ())),
                          preferred_element_type=jnp.float32)
        + b1c[...]
    )
    h1o[...] = h
    s1o[...] = jnp.sum(h, axis=1, keepdims=True)[None]
    ss1o[...] = jnp.sum(h * h, axis=1, keepdims=True)[None]
    wlo[...] = jnp.dot(ww[...], xwt[...], preferred_element_type=jnp.float32)


def _k2(h1, s1, ss1, g1c, beta1c, w2, b2c, h2o, s2o, ss2o):
    inv_b = jnp.float32(1.0 / B)
    mean = jnp.sum(s1[...], axis=0) * inv_b
    m2 = jnp.sum(ss1[...], axis=0) * inv_b
    var = m2 - mean * mean
    rstd = lax.rsqrt(var + EPS)
    hn = jnp.maximum((h1[...] - mean) * (rstd * g1c[...]) + beta1c[...], 0.0)
    h2 = jnp.dot(w2[...], hn, preferred_element_type=jnp.float32) + b2c[...]
    h2o[...] = h2
    s2o[...] = jnp.sum(h2, axis=1, keepdims=True)[None]
    ss2o[...] = jnp.sum(h2 * h2, axis=1, keepdims=True)[None]


def _k3(h2, s2, ss2, g2c, beta2c, w3, b3, bw, wl, out):
    inv_b = jnp.float32(1.0 / B)
    mean = jnp.sum(s2[...], axis=0) * inv_b
    m2 = jnp.sum(ss2[...], axis=0) * inv_b
    var = m2 - mean * mean
    rstd = lax.rsqrt(var + EPS)
    hn = jnp.maximum((h2[...] - mean) * (rstd * g2c[...]) + beta2c[...], 0.0)
    out[...] = (
        jnp.dot(w3[...], hn, preferred_element_type=jnp.float32)
        + b3[...]
        + bw[...]
        + wl[...]
    )


def kernel(x_num, x_wide, x_cat, tables, W1, b1, g1, beta1, W2, b2, g2, beta2, W3, b3, Ww, bw):
    tab_t = tables.transpose(0, 2, 1).reshape(F * D, V)
    xcat_t = x_cat.T
    xnt = x_num.T
    xwt = x_wide.T

    xemb3 = _sc_plane_gather(tab_t, xcat_t)

    w1T = W1.T
    w1eT = w1T[: F * D]
    w1nT = w1T[F * D :]

    h1, s1, ss1, wl = pl.pallas_call(
        _k1,
        grid=(NB,),
        in_specs=[
            pl.BlockSpec((F * D, BB), lambda i: (0, i)),
            pl.BlockSpec((NUM, BB), lambda i: (0, i)),
            pl.BlockSpec((WIDE, BB), lambda i: (0, i)),
            pl.BlockSpec((F * D, H1), lambda i: (0, 0)),
            pl.BlockSpec((NUM, H1), lambda i: (0, 0)),
            pl.BlockSpec((H1, 1), lambda i: (0, 0)),
            pl.BlockSpec((1, WIDE), lambda i: (0, 0)),
        ],
        out_specs=[
            pl.BlockSpec((H1, BB), lambda i: (0, i)),
            pl.BlockSpec((1, H1, 1), lambda i: (i, 0, 0)),
            pl.BlockSpec((1, H1, 1), lambda i: (i, 0, 0)),
            pl.BlockSpec((1, BB), lambda i: (0, i)),
        ],
        out_shape=[
            jax.ShapeDtypeStruct((H1, B), jnp.float32),
            jax.ShapeDtypeStruct((NB, H1, 1), jnp.float32),
            jax.ShapeDtypeStruct((NB, H1, 1), jnp.float32),
            jax.ShapeDtypeStruct((1, B), jnp.float32),
        ],
    )(xemb3, xnt, xwt, w1eT, w1nT, b1.reshape(H1, 1), Ww)

    h2, s2, ss2 = pl.pallas_call(
        _k2,
        grid=(NB,),
        in_specs=[
            pl.BlockSpec((H1, BB), lambda i: (0, i)),
            pl.BlockSpec((NB, H1, 1), lambda i: (0, 0, 0)),
            pl.BlockSpec((NB, H1, 1), lambda i: (0, 0, 0)),
            pl.BlockSpec((H1, 1), lambda i: (0, 0)),
            pl.BlockSpec((H1, 1), lambda i: (0, 0)),
            pl.BlockSpec((H2, H1), lambda i: (0, 0)),
            pl.BlockSpec((H2, 1), lambda i: (0, 0)),
        ],
        out_specs=[
            pl.BlockSpec((H2, BB), lambda i: (0, i)),
            pl.BlockSpec((1, H2, 1), lambda i: (i, 0, 0)),
            pl.BlockSpec((1, H2, 1), lambda i: (i, 0, 0)),
        ],
        out_shape=[
            jax.ShapeDtypeStruct((H2, B), jnp.float32),
            jax.ShapeDtypeStruct((NB, H2, 1), jnp.float32),
            jax.ShapeDtypeStruct((NB, H2, 1), jnp.float32),
        ],
    )(h1, s1, ss1, g1.reshape(H1, 1), beta1.reshape(H1, 1), W2, b2.reshape(H2, 1))

    out = pl.pallas_call(
        _k3,
        grid=(NB,),
        in_specs=[
            pl.BlockSpec((H2, BB), lambda i: (0, i)),
            pl.BlockSpec((NB, H2, 1), lambda i: (0, 0, 0)),
            pl.BlockSpec((NB, H2, 1), lambda i: (0, 0, 0)),
            pl.BlockSpec((H2, 1), lambda i: (0, 0)),
            pl.BlockSpec((H2, 1), lambda i: (0, 0)),
            pl.BlockSpec((1, H2), lambda i: (0, 0)),
            pl.BlockSpec((1, 1), lambda i: (0, 0)),
            pl.BlockSpec((1, 1), lambda i: (0, 0)),
            pl.BlockSpec((1, BB), lambda i: (0, i)),
        ],
        out_specs=pl.BlockSpec((1, BB), lambda i: (0, i)),
        out_shape=jax.ShapeDtypeStruct((1, B), jnp.float32),
    )(h2, s2, ss2, g2.reshape(H2, 1), beta2.reshape(H2, 1), W3,
      b3.reshape(1, 1), bw.reshape(1, 1), wl)

    return out[0]
